# baseline (speedup 1.0000x reference)
import functools

import jax
import jax.numpy as jnp
from jax import lax
from jax.experimental import pallas as pl
from jax.experimental.pallas import tpu as pltpu
from jax.experimental.pallas import tpu_sc as plsc

_EPS = 1e-5

_B = 16384
_D = 32
_P = 128
_NCOMBO = 128

_NC = 2
_NS = 16
_L = 16
_NW = _NC * _NS
_ROWS_W = _B // _NW
_CHROWS = 128
_CH = _ROWS_W // _CHROWS


def _table_body(d_ref, t_ref, a_ref, dt, dw, db, tt, tw, tb, at_, aw, ab_,
                w_ref, b_ref, out, combo_ref):
    nrow = _B // _CHROWS
    combo_ref[...] = (d_ref[...] * 4 + t_ref[...] * 2
                      + a_ref[...]).reshape(nrow, _CHROWS)
    dw, db = dw[...].reshape(1, _D), db[...].reshape(1, _D)
    tw, tb = tw[...].reshape(1, _D), tb[...].reshape(1, _D)
    aw, ab_ = aw[...].reshape(1, _D), ab_[...].reshape(1, _D)
    bb = b_ref[...].reshape(1, _P)
    def ln(x, wv, bv):
        mu = jnp.mean(x, axis=-1, keepdims=True)
        var = jnp.mean((x - mu) ** 2, axis=-1, keepdims=True)
        return (x - mu) * lax.rsqrt(var + _EPS) * wv + bv

    dn = ln(dt[...], dw, db)
    tn = ln(tt[...], tw, tb)
    an = ln(at_[...], aw, ab_)

    w = w_ref[...]
    dims = (((1,), (1,)), ((), ()))
    pd = lax.dot_general(dn, w[:, :_D], dims, preferred_element_type=jnp.float32)
    pt = lax.dot_general(tn, w[:, _D:2 * _D], dims, preferred_element_type=jnp.float32)
    pa = lax.dot_general(an, w[:, 2 * _D:], dims, preferred_element_type=jnp.float32)

    ic = lax.broadcasted_iota(jnp.int32, (_NCOMBO, _D), 0)
    iv = lax.broadcasted_iota(jnp.int32, (_NCOMBO, _D), 1)
    e_d = (ic // 4 == iv).astype(jnp.float32)
    hd = jnp.dot(e_d, pd, preferred_element_type=jnp.float32)

    cid = lax.broadcasted_iota(jnp.int32, (_NCOMBO, _P), 0)
    tbit = ((cid >> 1) & 1).astype(jnp.float32)
    abit = (cid & 1).astype(jnp.float32)
    h = (hd
         + pt[0:1, :] + tbit * (pt[1:2, :] - pt[0:1, :])
         + pa[0:1, :] + abit * (pa[1:2, :] - pa[0:1, :])
         + bb)
    out[...] = h * (1.0 / (1.0 + jnp.exp(-h)))


_table_call = pl.pallas_call(
    _table_body,
    out_shape=(
        jax.ShapeDtypeStruct((_NCOMBO, _P), jnp.float32),
        jax.ShapeDtypeStruct((_B // _CHROWS, _CHROWS), jnp.int32),
    ),
)


def _gather_body(combo_hbm, table_hbm, out_hbm,
                 idx_v, rows_v, table_sh, isem, tsem, gsems, ssem):
    sid = lax.axis_index("s")
    wid = sid * _NC + lax.axis_index("c")
    base = wid * _ROWS_W
    ic = pltpu.async_copy(combo_hbm.at[pl.ds(wid * _CH, _CH)], idx_v, isem)
    nstage = _NCOMBO // _NS
    tv = rows_v.at[pl.ds(0, nstage)]
    trows = pl.ds(sid * nstage, nstage)
    pltpu.async_copy(table_hbm.at[trows], tv, tsem).wait()
    pltpu.sync_copy(tv, table_sh.at[trows])
    ic.wait()
    plsc.subcore_barrier()
    gathers = [
        pltpu.async_copy(table_sh.at[idx_v.at[j]],
                         rows_v.at[pl.ds(j * _CHROWS, _CHROWS)], gsems.at[j])
        for j in range(_CH)
    ]
    stores = []
    for j in range(_CH):
        gathers[j].wait()
        stores.append(
            pltpu.async_copy(rows_v.at[pl.ds(j * _CHROWS, _CHROWS)],
                             out_hbm.at[pl.ds(base + j * _CHROWS, _CHROWS)],
                             ssem))
    for s in stores:
        s.wait()


@functools.cache
def _make_gather_call():
    mesh = plsc.VectorSubcoreMesh(core_axis_name="c", subcore_axis_name="s",
                                  num_cores=_NC, num_subcores=_NS)
    return pl.kernel(
        _gather_body,
        mesh=mesh,
        out_type=jax.ShapeDtypeStruct((_B, _P), jnp.float32),
        scratch_types=[
            pltpu.VMEM((_CH, _CHROWS), jnp.int32),
            pltpu.VMEM((_ROWS_W, _P), jnp.float32),
            pltpu.VMEM_SHARED((_NCOMBO, _P), jnp.float32),
            pltpu.SemaphoreType.DMA,
            pltpu.SemaphoreType.DMA,
            pltpu.SemaphoreType.DMA((_CH,)),
            pltpu.SemaphoreType.DMA,
        ],
    )


def kernel(layer_depth, layer_type, ab_type, depth_table, depth_ln_w, depth_ln_b,
           type_table, type_ln_w, type_ln_b, ab_table, ab_ln_w, ab_ln_b, W, b):
    table, combo = _table_call(
        layer_depth.astype(jnp.int32),
        layer_type.astype(jnp.int32),
        ab_type.astype(jnp.int32),
        depth_table, depth_ln_w, depth_ln_b,
        type_table, type_ln_w, type_ln_b,
        ab_table, ab_ln_w, ab_ln_b,
        W, b,
    )
    return _make_gather_call()(combo, table)


# speedup vs baseline: 1.0416x; 1.0416x over previous
import functools

import jax
import jax.numpy as jnp
from jax import lax
from jax.experimental import pallas as pl
from jax.experimental.pallas import tpu as pltpu
from jax.experimental.pallas import tpu_sc as plsc

_EPS = 1e-5

_B = 16384
_D = 32
_P = 128
_NCOMBO = 128

_NC = 2
_NS = 16
_L = 16
_NW = _NC * _NS
_ROWS_W = _B // _NW
_CHROWS = 128
_CH = _ROWS_W // _CHROWS


def _table_body(d_ref, t_ref, a_ref, dt, dw, db, tt, tw, tb, at_, aw, ab_,
                w_ref, b_ref, out, combo_ref):
    nrow = _B // _CHROWS
    combo_ref[...] = (d_ref[...] * 4 + t_ref[...] * 2
                      + a_ref[...]).reshape(nrow, _CHROWS)
    dw, db = dw[...].reshape(1, _D), db[...].reshape(1, _D)
    tw, tb = tw[...].reshape(1, _D), tb[...].reshape(1, _D)
    aw, ab_ = aw[...].reshape(1, _D), ab_[...].reshape(1, _D)
    bb = b_ref[...].reshape(1, _P)
    def ln(x, wv, bv):
        mu = jnp.mean(x, axis=-1, keepdims=True)
        var = jnp.mean((x - mu) ** 2, axis=-1, keepdims=True)
        return (x - mu) * lax.rsqrt(var + _EPS) * wv + bv

    dn = ln(dt[...], dw, db)
    tn = ln(tt[...], tw, tb)
    an = ln(at_[...], aw, ab_)

    wt = w_ref[...]
    pd = jnp.dot(dn, wt[:_D], preferred_element_type=jnp.float32)
    pt = jnp.dot(tn, wt[_D:2 * _D], preferred_element_type=jnp.float32)
    pa = jnp.dot(an, wt[2 * _D:], preferred_element_type=jnp.float32)

    ic = lax.broadcasted_iota(jnp.int32, (_NCOMBO, _D), 0)
    iv = lax.broadcasted_iota(jnp.int32, (_NCOMBO, _D), 1)
    e_d = (ic // 4 == iv).astype(jnp.float32)
    hd = jnp.dot(e_d, pd, preferred_element_type=jnp.float32)

    cid = lax.broadcasted_iota(jnp.int32, (_NCOMBO, _P), 0)
    tbit = ((cid >> 1) & 1).astype(jnp.float32)
    abit = (cid & 1).astype(jnp.float32)
    h = (hd
         + pt[0:1, :] + tbit * (pt[1:2, :] - pt[0:1, :])
         + pa[0:1, :] + abit * (pa[1:2, :] - pa[0:1, :])
         + bb)
    out[...] = h * (1.0 / (1.0 + jnp.exp(-h)))


_table_call = pl.pallas_call(
    _table_body,
    out_shape=(
        jax.ShapeDtypeStruct((_NCOMBO, _P), jnp.float32),
        jax.ShapeDtypeStruct((_B // _CHROWS, _CHROWS), jnp.int32),
    ),
)


def _gather_body(combo_hbm, table_hbm, out_hbm,
                 idx_v, rows_v, table_sh, isem, tsem, gsems, ssem):
    sid = lax.axis_index("s")
    wid = sid * _NC + lax.axis_index("c")
    base = wid * _ROWS_W
    ic = pltpu.async_copy(combo_hbm.at[pl.ds(wid * _CH, _CH)], idx_v, isem)
    nstage = _NCOMBO // _NS
    tv = rows_v.at[pl.ds(0, nstage)]
    trows = pl.ds(sid * nstage, nstage)
    pltpu.async_copy(table_hbm.at[trows], tv, tsem).wait()
    pltpu.sync_copy(tv, table_sh.at[trows])
    ic.wait()
    plsc.subcore_barrier()
    gathers = [
        pltpu.async_copy(table_sh.at[idx_v.at[j]],
                         rows_v.at[pl.ds(j * _CHROWS, _CHROWS)], gsems.at[j])
        for j in range(_CH)
    ]
    stores = []
    for j in range(_CH):
        gathers[j].wait()
        stores.append(
            pltpu.async_copy(rows_v.at[pl.ds(j * _CHROWS, _CHROWS)],
                             out_hbm.at[pl.ds(base + j * _CHROWS, _CHROWS)],
                             ssem))
    for s in stores:
        s.wait()


@functools.cache
def _make_gather_call():
    mesh = plsc.VectorSubcoreMesh(core_axis_name="c", subcore_axis_name="s",
                                  num_cores=_NC, num_subcores=_NS)
    return pl.kernel(
        _gather_body,
        mesh=mesh,
        out_type=jax.ShapeDtypeStruct((_B, _P), jnp.float32),
        scratch_types=[
            pltpu.VMEM((_CH, _CHROWS), jnp.int32),
            pltpu.VMEM((_ROWS_W, _P), jnp.float32),
            pltpu.VMEM_SHARED((_NCOMBO, _P), jnp.float32),
            pltpu.SemaphoreType.DMA,
            pltpu.SemaphoreType.DMA,
            pltpu.SemaphoreType.DMA((_CH,)),
            pltpu.SemaphoreType.DMA,
        ],
    )


def kernel(layer_depth, layer_type, ab_type, depth_table, depth_ln_w, depth_ln_b,
           type_table, type_ln_w, type_ln_b, ab_table, ab_ln_w, ab_ln_b, W, b):
    table, combo = _table_call(
        layer_depth.astype(jnp.int32),
        layer_type.astype(jnp.int32),
        ab_type.astype(jnp.int32),
        depth_table, depth_ln_w, depth_ln_b,
        type_table, type_ln_w, type_ln_b,
        ab_table, ab_ln_w, ab_ln_b,
        W.T, b,
    )
    return _make_gather_call()(combo, table)
